# baseline (speedup 1.0000x reference)
import functools

import jax
import jax.numpy as jnp
from jax import lax
from jax.experimental import pallas as pl
from jax.experimental.pallas import tpu as pltpu
from jax.experimental.pallas import tpu_sc as plsc

K = 32


def _encode_body(x_ref, db_ref, w_ref, eb_ref, out_ref):
    xb = x_ref[...]
    lrep = xb.shape[0] // db_ref.shape[0]
    db = jnp.broadcast_to(db_ref[...][None], (lrep,) + db_ref.shape)
    db = db.reshape(xb.shape)
    xc = xb - db
    acc = jax.lax.dot_general(
        xc, w_ref[...], (((1,), (0,)), ((), ())),
        precision=jax.lax.Precision.DEFAULT,
        preferred_element_type=jnp.float32,
    )
    out_ref[...] = acc + eb_ref[...]


def _encode(x2, decoder_b, we_t, encoder_b, *, rb=768, hb=2048):
    T, C = x2.shape
    H = we_t.shape[1]
    L = decoder_b.shape[0]
    eb2 = encoder_b.reshape(1, H)
    grid = (T // rb, H // hb)
    return pl.pallas_call(
        _encode_body,
        grid=grid,
        in_specs=[
            pl.BlockSpec((rb, C), lambda r, h: (r, 0)),
            pl.BlockSpec((L, C), lambda r, h: (0, 0)),
            pl.BlockSpec((C, hb), lambda r, h: (0, h)),
            pl.BlockSpec((1, hb), lambda r, h: (0, h)),
        ],
        out_specs=pl.BlockSpec((rb, hb), lambda r, h: (r, h)),
        out_shape=jax.ShapeDtypeStruct((T, H), jnp.float32),
    )(x2, decoder_b, we_t, eb2)


def _topk_body(pre_ref, vals_ref, idx_ref, scratch):
    scratch[...] = pre_ref[...]
    vals_ref[...] = jnp.zeros_like(vals_ref)
    idx_ref[...] = jnp.zeros_like(idx_ref)
    R, H = scratch.shape

    def body(k, _):
        p = scratch[...]
        m = jnp.max(p, axis=1, keepdims=True)
        lane = jax.lax.broadcasted_iota(jnp.int32, p.shape, 1)
        j = jnp.min(jnp.where(p == m, lane, H), axis=1, keepdims=True)
        onehot = jax.lax.broadcasted_iota(jnp.int32, (R, K), 1) == k
        vals_ref[...] += jnp.where(onehot, m, 0.0)
        idx_ref[...] += jnp.where(onehot, j, 0)
        scratch[...] = jnp.where(lane == j, -jnp.inf, p)
        return 0

    lax.fori_loop(0, K, body, 0)


def _topk_tc(pre, *, rb=128):
    T, H = pre.shape
    return pl.pallas_call(
        _topk_body,
        grid=(T // rb,),
        in_specs=[pl.BlockSpec((rb, H), lambda r: (r, 0))],
        out_specs=[
            pl.BlockSpec((rb, K), lambda r: (r, 0)),
            pl.BlockSpec((rb, K), lambda r: (r, 0)),
        ],
        out_shape=[
            jax.ShapeDtypeStruct((T, K), jnp.float32),
            jax.ShapeDtypeStruct((T, K), jnp.int32),
        ],
        scratch_shapes=[pltpu.VMEM((rb, H), jnp.float32)],
    )(pre)


_NEG = -3.0e38


def _splat16(vec, lane):
    dn = lax.GatherDimensionNumbers(
        offset_dims=(), collapsed_slice_dims=(0,), start_index_map=(0,))
    idx = jnp.full((16, 1), lane, jnp.int32)
    return lax.gather(vec, idx, dn, (1,),
                      mode=lax.GatherScatterMode.PROMISE_IN_BOUNDS)


def _rev16(v):
    return lax.rev(v, dimensions=(0,))


def _merge_run(run, v):
    return jnp.sort(jnp.maximum(run, _rev16(jnp.sort(v))))


def _merge_sorted(a, b):
    return jnp.sort(jnp.maximum(a, _rev16(b)))


def _top16_static(read_fn, n):
    neg = jnp.full((16,), _NEG, jnp.float32)

    def step(s, runs):
        return tuple(
            _merge_run(runs[c], read_fn(4 * s + c)) for c in range(4))

    r0, r1, r2, r3 = lax.fori_loop(0, n // 4, step, (neg, neg, neg, neg))
    return _merge_sorted(_merge_sorted(r0, r1), _merge_sorted(r2, r3))


def _top16_dyn(read_fn, nv):
    neg = jnp.full((16,), _NEG, jnp.float32)
    return lax.fori_loop(0, nv, lambda s, r: _merge_run(r, read_fn(s)), neg)


def _merge_run_kv(rk, rv, k, v):
    sk, sv = plsc.sort_key_val(k, v)
    sk, sv = _rev16(sk), _rev16(sv)
    m = rk >= sk
    ok, ov = plsc.sort_key_val(jnp.where(m, rk, sk), jnp.where(m, rv, sv))
    return ok, ov


def _merge_sorted_kv(ak, av, bk, bv):
    bk, bv = _rev16(bk), _rev16(bv)
    m = ak >= bk
    ok, ov = plsc.sort_key_val(jnp.where(m, ak, bk), jnp.where(m, av, bv))
    return ok, ov


def _top16_kv(read_fn, n, nchain):
    neg = jnp.full((16,), _NEG, jnp.float32)
    zero = jnp.zeros((16,), jnp.int32)

    def step(s, runs):
        out = []
        for c in range(nchain):
            k, v = read_fn(nchain * s + c)
            out.append(tuple(_merge_run_kv(runs[c][0], runs[c][1], k, v)))
        return tuple(out)

    runs = lax.fori_loop(0, n // nchain, step, ((neg, zero),) * nchain)
    while len(runs) > 1:
        runs = tuple(
            tuple(_merge_sorted_kv(*runs[2 * i], *runs[2 * i + 1]))
            for i in range(len(runs) // 2))
    return runs[0]


def _make_topk_sc(T, H, NC, NS):
    NW = NC * NS
    tpw = T // NW
    assert tpw % 2 == 0 and tpw >= 4
    NGR = H // 256
    NB = H // 16

    mesh = plsc.VectorSubcoreMesh(core_axis_name="c", subcore_axis_name="s")

    @functools.partial(
        pl.kernel,
        mesh=mesh,
        out_type=[
            jax.ShapeDtypeStruct((T * K,), jnp.float32),
            jax.ShapeDtypeStruct((T * K,), jnp.int32),
        ],
        scratch_types=[
            pltpu.VMEM((2, H), jnp.float32),
            pltpu.VMEM((NB,), jnp.float32),
            pltpu.VMEM((NB,), jnp.float32),
            pltpu.VMEM((2 * K * 16,), jnp.float32),
            pltpu.VMEM((2 * K * 16,), jnp.int32),
            pltpu.VMEM((2 * K * 16,), jnp.float32),
            pltpu.VMEM((tpw * K,), jnp.float32),
            pltpu.VMEM((tpw * K,), jnp.int32),
            pltpu.SemaphoreType.DMA,
            pltpu.SemaphoreType.DMA,
        ],
        compiler_params=pltpu.CompilerParams(needs_layout_passes=False),
    )
    def topk(pre_hbm, vals_hbm, idx_hbm,
             row_v, gm_v, gm2_v, cv_v, ci_v, cv2_v,
             ov_v, oi_v, gsem0, gsem1):
        wid = lax.axis_index("s") * NC + lax.axis_index("c")
        base = wid * tpw
        iota = lax.iota(jnp.int32, 16)

        def fetch(i, buf, sem):
            pltpu.async_copy(pre_hbm.at[base + i], row_v.at[buf], sem)

        def fwait(buf, sem):
            pltpu.make_async_copy(
                pre_hbm.at[base], row_v.at[buf], sem).wait()

        def process(i, buf):
            def grp(g, _):
                m = jnp.full((16,), _NEG, jnp.float32)
                for j in range(16):
                    m = jnp.maximum(
                        m, row_v[buf, pl.ds(g * 256 + j * 16, 16)])
                gm_v[pl.ds(g * 16, 16)] = m
                return 0

            lax.fori_loop(0, NGR, grp, 0)

            def rd1(s):
                return gm_v[pl.ds(s * 16, 16)], iota + s * 16

            r1k, r1v = _top16_kv(rd1, NB // 16, 4)
            t16 = _splat16(r1k, 0)

            def rem(g, _):
                v = gm_v[pl.ds(g * 16, 16)]
                gm2_v[pl.ds(g * 16, 16)] = jnp.where(v >= t16, _NEG, v)
                return 0

            lax.fori_loop(0, NB // 16, rem, 0)

            def rd2(s):
                return gm2_v[pl.ds(s * 16, 16)], iota + s * 16

            r2k, r2v = _top16_kv(rd2, NB // 16, 4)

            bufidx = jnp.full((16,), buf, jnp.int32)
            for rr in range(2 * 16):
                bidv = r1v if rr < 16 else r2v
                bid = _splat16(bidv, rr % 16)
                eidx = (jnp.right_shift(bid, 4) * 256
                        + jnp.bitwise_and(bid, 15) + iota * 16)
                candv = plsc.load_gather(row_v, [bufidx, eidx])
                cv_v[pl.ds(rr * 16, 16)] = candv
                ci_v[pl.ds(rr * 16, 16)] = eidx

            def rd3(s):
                return (cv_v[pl.ds(s * 16, 16)], ci_v[pl.ds(s * 16, 16)])

            f1k, f1v = _top16_kv(rd3, 2 * 16, 2)
            tf = _splat16(f1k, 0)

            def rem2(s, _):
                v = cv_v[pl.ds(s * 16, 16)]
                cv2_v[pl.ds(s * 16, 16)] = jnp.where(v >= tf, _NEG, v)
                return 0

            lax.fori_loop(0, 2 * 16, rem2, 0)

            def rd4(s):
                return (cv2_v[pl.ds(s * 16, 16)], ci_v[pl.ds(s * 16, 16)])

            f2k, f2v = _top16_kv(rd4, 2 * 16, 2)

            ov_v[pl.ds(i * K, 16)] = f1k
            ov_v[pl.ds(i * K + 16, 16)] = f2k
            oi_v[pl.ds(i * K, 16)] = f1v
            oi_v[pl.ds(i * K + 16, 16)] = f2v

        fetch(0, 0, gsem0)

        def pair(p, _):
            i0 = 2 * p
            fetch(i0 + 1, 1, gsem1)
            fwait(0, gsem0)
            process(i0, 0)

            @pl.when(i0 + 2 < tpw)
            def _():
                fetch(i0 + 2, 0, gsem0)

            fwait(1, gsem1)
            process(i0 + 1, 1)
            return 0

        lax.fori_loop(0, tpw // 2, pair, 0)
        pltpu.sync_copy(ov_v, vals_hbm.at[pl.ds(base * K, tpw * K)])
        pltpu.sync_copy(oi_v, idx_hbm.at[pl.ds(base * K, tpw * K)])

    return topk


def _make_decode(T, C, L, NC, NS):
    NW = NC * NS
    tpw = T // NW
    assert tpw % 2 == 0 and tpw >= 4
    LANES = 16
    nch = C // LANES

    mesh = plsc.VectorSubcoreMesh(core_axis_name="c", subcore_axis_name="s")

    @functools.partial(
        pl.kernel,
        mesh=mesh,
        out_type=jax.ShapeDtypeStruct((T, C), jnp.float32),
        scratch_types=[
            pltpu.VMEM((tpw, K), jnp.int32),
            pltpu.VMEM((tpw * K,), jnp.float32),
            pltpu.VMEM((L, C), jnp.float32),
            pltpu.VMEM((2, K, C), jnp.float32),
            pltpu.VMEM((2, C), jnp.float32),
            pltpu.SemaphoreType.DMA,
            pltpu.SemaphoreType.DMA,
            pltpu.SemaphoreType.DMA,
            pltpu.SemaphoreType.DMA,
        ],
    )
    def decode(vals_hbm, idx_hbm, wdt_hbm, db_hbm, out_hbm,
               idx_v, vals_v, db_v, rows_v, out_v, gsem0, gsem1, osem0, osem1):
        wid = lax.axis_index("s") * NC + lax.axis_index("c")
        base = wid * tpw
        pltpu.sync_copy(idx_hbm.at[pl.ds(base, tpw)], idx_v)
        pltpu.sync_copy(vals_hbm.at[pl.ds(base * K, tpw * K)], vals_v)
        pltpu.sync_copy(db_hbm, db_v)

        def gather(i, buf, sem):
            pltpu.async_copy(wdt_hbm.at[idx_v.at[i]], rows_v.at[buf], sem)

        def gwait(buf, sem):
            pltpu.make_async_copy(
                wdt_hbm.at[idx_v.at[0]], rows_v.at[buf], sem).wait()

        def owait(buf, i, sem):
            pltpu.make_async_copy(
                out_v.at[buf], out_hbm.at[i], sem).wait()

        def compute(i, buf):
            lrow = lax.rem(i, L)
            dn = lax.GatherDimensionNumbers(
                offset_dims=(), collapsed_slice_dims=(0,),
                start_index_map=(0,))

            def _splat(vec, k):
                idx = jnp.full((LANES, 1), k, jnp.int32)
                return lax.gather(
                    vec, idx, dn, (1,),
                    mode=lax.GatherScatterMode.PROMISE_IN_BOUNDS)

            splats = []
            for k0 in range(0, K, LANES):
                vchunk = jnp.maximum(vals_v[pl.ds(i * K + k0, LANES)], 0.0)
                for k in range(LANES):
                    splats.append(_splat(vchunk, k))

            def col(c, _):
                sl = pl.ds(c * LANES, LANES)
                acc = db_v[lrow, sl]
                for k in range(K):
                    acc = acc + splats[k] * rows_v[buf, k, sl]
                out_v[buf, sl] = acc
                return 0

            lax.fori_loop(0, nch, col, 0)

        gather(0, 0, gsem0)

        def pair(p, _):
            i0 = 2 * p
            gather(i0 + 1, 1, gsem1)
            gwait(0, gsem0)

            @pl.when(i0 >= 2)
            def _():
                owait(0, base + i0 - 2, osem0)

            compute(i0, 0)
            pltpu.async_copy(out_v.at[0], out_hbm.at[base + i0], osem0)

            @pl.when(i0 + 2 < tpw)
            def _():
                gather(i0 + 2, 0, gsem0)

            gwait(1, gsem1)

            @pl.when(i0 >= 2)
            def _():
                owait(1, base + i0 - 1, osem1)

            compute(i0 + 1, 1)
            pltpu.async_copy(out_v.at[1], out_hbm.at[base + i0 + 1], osem1)
            return 0

        lax.fori_loop(0, tpw // 2, pair, 0)
        owait(0, base + tpw - 2, osem0)
        owait(1, base + tpw - 1, osem1)

    return decode


def kernel(x, decoder_b, encoder_w, encoder_b, decoder_w):
    B, L, C = x.shape
    H = encoder_w.shape[0]
    T = B * L

    x2 = x.reshape(T, C)
    we_t = encoder_w.T
    wd_t = decoder_w.T

    info = plsc.get_sparse_core_info()
    NHALF = 4
    Th = T // NHALF
    topk = _make_topk_sc(Th, H, info.num_cores, info.num_subcores)
    decode = _make_decode(Th, C, L, info.num_cores, info.num_subcores)

    stage = []
    for h in range(NHALF):
        xh = lax.slice_in_dim(x2, h * Th, (h + 1) * Th, axis=0)
        pre = _encode(xh, decoder_b, we_t, encoder_b)
        stage.append(topk(pre))
    outs = []
    for h in range(NHALF):
        vals_f, idx_f = stage[h]
        outs.append(decode(vals_f, idx_f.reshape(Th, K), wd_t, decoder_b))
    out2 = jnp.concatenate(outs, axis=0)
    return out2.reshape(B, L, C)


# speedup vs baseline: 1.0252x; 1.0252x over previous
import functools

import jax
import jax.numpy as jnp
from jax import lax
from jax.experimental import pallas as pl
from jax.experimental.pallas import tpu as pltpu
from jax.experimental.pallas import tpu_sc as plsc

K = 32


def _encode_body(x_ref, db_ref, w_ref, eb_ref, out_ref):
    xb = x_ref[...]
    lrep = xb.shape[0] // db_ref.shape[0]
    db = jnp.broadcast_to(db_ref[...][None], (lrep,) + db_ref.shape)
    db = db.reshape(xb.shape)
    xc = xb - db
    acc = jax.lax.dot_general(
        xc, w_ref[...], (((1,), (0,)), ((), ())),
        precision=jax.lax.Precision.DEFAULT,
        preferred_element_type=jnp.float32,
    )
    out_ref[...] = acc + eb_ref[...]


def _encode(x2, decoder_b, we_t, encoder_b, *, rb=768, hb=2048):
    T, C = x2.shape
    H = we_t.shape[1]
    L = decoder_b.shape[0]
    eb2 = encoder_b.reshape(1, H)
    grid = (T // rb, H // hb)
    return pl.pallas_call(
        _encode_body,
        grid=grid,
        in_specs=[
            pl.BlockSpec((rb, C), lambda r, h: (r, 0)),
            pl.BlockSpec((L, C), lambda r, h: (0, 0)),
            pl.BlockSpec((C, hb), lambda r, h: (0, h)),
            pl.BlockSpec((1, hb), lambda r, h: (0, h)),
        ],
        out_specs=pl.BlockSpec((rb, hb), lambda r, h: (r, h)),
        out_shape=jax.ShapeDtypeStruct((T, H), jnp.float32),
    )(x2, decoder_b, we_t, eb2)


def _topk_body(pre_ref, vals_ref, idx_ref, scratch):
    scratch[...] = pre_ref[...]
    vals_ref[...] = jnp.zeros_like(vals_ref)
    idx_ref[...] = jnp.zeros_like(idx_ref)
    R, H = scratch.shape

    def body(k, _):
        p = scratch[...]
        m = jnp.max(p, axis=1, keepdims=True)
        lane = jax.lax.broadcasted_iota(jnp.int32, p.shape, 1)
        j = jnp.min(jnp.where(p == m, lane, H), axis=1, keepdims=True)
        onehot = jax.lax.broadcasted_iota(jnp.int32, (R, K), 1) == k
        vals_ref[...] += jnp.where(onehot, m, 0.0)
        idx_ref[...] += jnp.where(onehot, j, 0)
        scratch[...] = jnp.where(lane == j, -jnp.inf, p)
        return 0

    lax.fori_loop(0, K, body, 0)


def _topk_tc(pre, *, rb=128):
    T, H = pre.shape
    return pl.pallas_call(
        _topk_body,
        grid=(T // rb,),
        in_specs=[pl.BlockSpec((rb, H), lambda r: (r, 0))],
        out_specs=[
            pl.BlockSpec((rb, K), lambda r: (r, 0)),
            pl.BlockSpec((rb, K), lambda r: (r, 0)),
        ],
        out_shape=[
            jax.ShapeDtypeStruct((T, K), jnp.float32),
            jax.ShapeDtypeStruct((T, K), jnp.int32),
        ],
        scratch_shapes=[pltpu.VMEM((rb, H), jnp.float32)],
    )(pre)


_NEG = -3.0e38


def _splat16(vec, lane):
    dn = lax.GatherDimensionNumbers(
        offset_dims=(), collapsed_slice_dims=(0,), start_index_map=(0,))
    idx = jnp.full((16, 1), lane, jnp.int32)
    return lax.gather(vec, idx, dn, (1,),
                      mode=lax.GatherScatterMode.PROMISE_IN_BOUNDS)


def _rev16(v):
    return lax.rev(v, dimensions=(0,))


def _merge_run(run, v):
    return jnp.sort(jnp.maximum(run, _rev16(jnp.sort(v))))


def _merge_sorted(a, b):
    return jnp.sort(jnp.maximum(a, _rev16(b)))


def _top16_static(read_fn, n):
    neg = jnp.full((16,), _NEG, jnp.float32)

    def step(s, runs):
        return tuple(
            _merge_run(runs[c], read_fn(4 * s + c)) for c in range(4))

    r0, r1, r2, r3 = lax.fori_loop(0, n // 4, step, (neg, neg, neg, neg))
    return _merge_sorted(_merge_sorted(r0, r1), _merge_sorted(r2, r3))


def _top16_dyn(read_fn, nv):
    neg = jnp.full((16,), _NEG, jnp.float32)
    return lax.fori_loop(0, nv, lambda s, r: _merge_run(r, read_fn(s)), neg)


def _merge_run_kv(rk, rv, k, v):
    sk, sv = plsc.sort_key_val(k, v)
    sk, sv = _rev16(sk), _rev16(sv)
    m = rk >= sk
    ok, ov = plsc.sort_key_val(jnp.where(m, rk, sk), jnp.where(m, rv, sv))
    return ok, ov


def _merge_sorted_kv(ak, av, bk, bv):
    bk, bv = _rev16(bk), _rev16(bv)
    m = ak >= bk
    ok, ov = plsc.sort_key_val(jnp.where(m, ak, bk), jnp.where(m, av, bv))
    return ok, ov


def _top16_kv(read_fn, n, nchain):
    neg = jnp.full((16,), _NEG, jnp.float32)
    zero = jnp.zeros((16,), jnp.int32)

    def step(s, runs):
        out = []
        for c in range(nchain):
            k, v = read_fn(nchain * s + c)
            out.append(tuple(_merge_run_kv(runs[c][0], runs[c][1], k, v)))
        return tuple(out)

    runs = lax.fori_loop(0, n // nchain, step, ((neg, zero),) * nchain)
    while len(runs) > 1:
        runs = tuple(
            tuple(_merge_sorted_kv(*runs[2 * i], *runs[2 * i + 1]))
            for i in range(len(runs) // 2))
    return runs[0]


def _make_topk_sc(T, H, NC, NS):
    NW = NC * NS
    tpw = T // NW
    assert tpw % 2 == 0 and tpw >= 4
    NGR = H // 256
    NB = H // 16

    mesh = plsc.VectorSubcoreMesh(core_axis_name="c", subcore_axis_name="s")

    @functools.partial(
        pl.kernel,
        mesh=mesh,
        out_type=[
            jax.ShapeDtypeStruct((T * K,), jnp.float32),
            jax.ShapeDtypeStruct((T * K,), jnp.int32),
        ],
        scratch_types=[
            pltpu.VMEM((2, H), jnp.float32),
            pltpu.VMEM((NB,), jnp.float32),
            pltpu.VMEM((NB,), jnp.float32),
            pltpu.VMEM((2 * K * 16,), jnp.float32),
            pltpu.VMEM((2 * K * 16,), jnp.int32),
            pltpu.VMEM((2 * K * 16,), jnp.float32),
            pltpu.VMEM((tpw * K,), jnp.float32),
            pltpu.VMEM((tpw * K,), jnp.int32),
            pltpu.SemaphoreType.DMA,
            pltpu.SemaphoreType.DMA,
        ],
        compiler_params=pltpu.CompilerParams(needs_layout_passes=False),
    )
    def topk(pre_hbm, vals_hbm, idx_hbm,
             row_v, gm_v, gm2_v, cv_v, ci_v, cv2_v,
             ov_v, oi_v, gsem0, gsem1):
        wid = lax.axis_index("s") * NC + lax.axis_index("c")
        base = wid * tpw
        iota = lax.iota(jnp.int32, 16)

        def fetch(i, buf, sem):
            pltpu.async_copy(pre_hbm.at[base + i], row_v.at[buf], sem)

        def fwait(buf, sem):
            pltpu.make_async_copy(
                pre_hbm.at[base], row_v.at[buf], sem).wait()

        def process(i, buf):
            def grp(g, _):
                m = jnp.full((16,), _NEG, jnp.float32)
                for j in range(16):
                    m = jnp.maximum(
                        m, row_v[buf, pl.ds(g * 256 + j * 16, 16)])
                gm_v[pl.ds(g * 16, 16)] = m
                return 0

            lax.fori_loop(0, NGR, grp, 0)

            def rd1(s):
                return gm_v[pl.ds(s * 16, 16)], iota + s * 16

            r1k, r1v = _top16_kv(rd1, NB // 16, 4)
            t16 = _splat16(r1k, 0)

            def rem(g, _):
                v = gm_v[pl.ds(g * 16, 16)]
                gm2_v[pl.ds(g * 16, 16)] = jnp.where(v >= t16, _NEG, v)
                return 0

            lax.fori_loop(0, NB // 16, rem, 0)

            def rd2(s):
                return gm2_v[pl.ds(s * 16, 16)], iota + s * 16

            r2k, r2v = _top16_kv(rd2, NB // 16, 4)

            bufidx = jnp.full((16,), buf, jnp.int32)
            for rr in range(2 * 16):
                bidv = r1v if rr < 16 else r2v
                bid = _splat16(bidv, rr % 16)
                eidx = (jnp.right_shift(bid, 4) * 256
                        + jnp.bitwise_and(bid, 15) + iota * 16)
                candv = plsc.load_gather(row_v, [bufidx, eidx])
                cv_v[pl.ds(rr * 16, 16)] = candv
                ci_v[pl.ds(rr * 16, 16)] = eidx

            def rd3(s):
                return (cv_v[pl.ds(s * 16, 16)], ci_v[pl.ds(s * 16, 16)])

            f1k, f1v = _top16_kv(rd3, 2 * 16, 2)
            tf = _splat16(f1k, 0)

            def rem2(s, _):
                v = cv_v[pl.ds(s * 16, 16)]
                cv2_v[pl.ds(s * 16, 16)] = jnp.where(v >= tf, _NEG, v)
                return 0

            lax.fori_loop(0, 2 * 16, rem2, 0)

            def rd4(s):
                return (cv2_v[pl.ds(s * 16, 16)], ci_v[pl.ds(s * 16, 16)])

            f2k, f2v = _top16_kv(rd4, 2 * 16, 2)

            ov_v[pl.ds(i * K, 16)] = f1k
            ov_v[pl.ds(i * K + 16, 16)] = f2k
            oi_v[pl.ds(i * K, 16)] = f1v
            oi_v[pl.ds(i * K + 16, 16)] = f2v

        fetch(0, 0, gsem0)

        def pair(p, _):
            i0 = 2 * p
            fetch(i0 + 1, 1, gsem1)
            fwait(0, gsem0)
            process(i0, 0)

            @pl.when(i0 + 2 < tpw)
            def _():
                fetch(i0 + 2, 0, gsem0)

            fwait(1, gsem1)
            process(i0 + 1, 1)
            return 0

        lax.fori_loop(0, tpw // 2, pair, 0)
        pltpu.sync_copy(ov_v, vals_hbm.at[pl.ds(base * K, tpw * K)])
        pltpu.sync_copy(oi_v, idx_hbm.at[pl.ds(base * K, tpw * K)])

    return topk


def _make_fused_sc(T, H, C, L, NC, NS):
    NW = NC * NS
    tpw = T // NW
    assert tpw % 2 == 0 and tpw >= 4
    NGR = H // 256
    NB = H // 16
    NR = 2 * 16
    LANES = 16
    nch = C // LANES

    mesh = plsc.VectorSubcoreMesh(core_axis_name="c", subcore_axis_name="s")

    @functools.partial(
        pl.kernel,
        mesh=mesh,
        out_type=jax.ShapeDtypeStruct((T, C), jnp.float32),
        scratch_types=[
            pltpu.VMEM((2, H), jnp.float32),
            pltpu.VMEM((NB,), jnp.float32),
            pltpu.VMEM((NB,), jnp.float32),
            pltpu.VMEM((NR * 16,), jnp.float32),
            pltpu.VMEM((NR * 16,), jnp.int32),
            pltpu.VMEM((NR * 16,), jnp.float32),
            pltpu.VMEM((2, 2 * K), jnp.float32),
            pltpu.VMEM((2, K), jnp.int32),
            pltpu.VMEM((2, K, C), jnp.float32),
            pltpu.VMEM((2, C), jnp.float32),
            pltpu.SemaphoreType.DMA,
            pltpu.SemaphoreType.DMA,
            pltpu.SemaphoreType.DMA,
            pltpu.SemaphoreType.DMA,
            pltpu.SemaphoreType.DMA,
            pltpu.SemaphoreType.DMA,
        ],
        compiler_params=pltpu.CompilerParams(needs_layout_passes=False),
    )
    def fused(pre_hbm, wde_hbm, out_hbm,
              row_v, gm_v, gm2_v, cv_v, ci_v, cv2_v, vv_v, di_v, rows_v,
              out_v, gsem0, gsem1, dsem0, dsem1, osem0, osem1):
        wid = lax.axis_index("s") * NC + lax.axis_index("c")
        base = wid * tpw
        iota = lax.iota(jnp.int32, 16)

        def fetch(i, buf, sem):
            pltpu.async_copy(pre_hbm.at[base + i], row_v.at[buf], sem)

        def fwait(buf, sem):
            pltpu.make_async_copy(
                pre_hbm.at[base], row_v.at[buf], sem).wait()

        def topk_row(buf):
            def grp(g, _):
                m = jnp.full((16,), _NEG, jnp.float32)
                for j in range(16):
                    m = jnp.maximum(
                        m, row_v[buf, pl.ds(g * 256 + j * 16, 16)])
                gm_v[pl.ds(g * 16, 16)] = m
                return 0

            lax.fori_loop(0, NGR, grp, 0)

            def rd1(s):
                return gm_v[pl.ds(s * 16, 16)], iota + s * 16

            r1k, r1v = _top16_kv(rd1, NB // 16, 4)
            t16 = _splat16(r1k, 0)

            def rem(g, _):
                v = gm_v[pl.ds(g * 16, 16)]
                gm2_v[pl.ds(g * 16, 16)] = jnp.where(v >= t16, _NEG, v)
                return 0

            lax.fori_loop(0, NB // 16, rem, 0)

            def rd2(s):
                return gm2_v[pl.ds(s * 16, 16)], iota + s * 16

            r2k, r2v = _top16_kv(rd2, NB // 16, 4)

            bufidx = jnp.full((16,), buf, jnp.int32)
            for rr in range(NR):
                bidv = r1v if rr < 16 else r2v
                bid = _splat16(bidv, rr % 16)
                eidx = (jnp.right_shift(bid, 4) * 256
                        + jnp.bitwise_and(bid, 15) + iota * 16)
                cv_v[pl.ds(rr * 16, 16)] = plsc.load_gather(
                    row_v, [bufidx, eidx])
                ci_v[pl.ds(rr * 16, 16)] = eidx

            def rd3(s):
                return (cv_v[pl.ds(s * 16, 16)], ci_v[pl.ds(s * 16, 16)])

            f1k, f1v = _top16_kv(rd3, NR, 2)
            tf = _splat16(f1k, 0)

            def rem2(s, _):
                v = cv_v[pl.ds(s * 16, 16)]
                cv2_v[pl.ds(s * 16, 16)] = jnp.where(v >= tf, _NEG, v)
                return 0

            lax.fori_loop(0, NR, rem2, 0)

            def rd4(s):
                return (cv2_v[pl.ds(s * 16, 16)], ci_v[pl.ds(s * 16, 16)])

            f2k, f2v = _top16_kv(rd4, NR, 2)
            return f1k, f1v, f2k, f2v

        def stage_and_gather(i, slot, dsem):
            f1k, f1v, f2k, f2v = topk_row(slot)
            vv_v[slot, pl.ds(0, 16)] = jnp.maximum(f1k, 0.0)
            vv_v[slot, pl.ds(16, 16)] = jnp.maximum(f2k, 0.0)
            di_v[slot, pl.ds(0, 16)] = f1v
            di_v[slot, pl.ds(16, 16)] = f2v
            pltpu.async_copy(
                wde_hbm.at[di_v.at[slot]], rows_v.at[slot], dsem)

        def decode_row(i, slot, dsem, osem):
            pltpu.make_async_copy(
                wde_hbm.at[di_v.at[slot]], rows_v.at[slot], dsem).wait()

            @pl.when(i >= 2)
            def _():
                pltpu.make_async_copy(
                    out_v.at[slot], out_hbm.at[base], osem).wait()

            v0 = vv_v[slot, pl.ds(0, 16)]
            v1 = vv_v[slot, pl.ds(16, 16)]
            splats = [_splat16(v0, k) for k in range(16)]
            splats += [_splat16(v1, k) for k in range(16)]

            def col(c, _):
                sl = pl.ds(c * LANES, LANES)
                acc = splats[0] * rows_v[slot, 0, sl]
                for k in range(1, K):
                    acc = acc + splats[k] * rows_v[slot, k, sl]
                out_v[slot, sl] = acc
                return 0

            lax.fori_loop(0, nch, col, 0)
            pltpu.async_copy(out_v.at[slot], out_hbm.at[base + i], osem)

        fetch(0, 0, gsem0)

        def pair(p, _):
            i0 = 2 * p
            fetch(i0 + 1, 1, gsem1)
            fwait(0, gsem0)
            stage_and_gather(i0, 0, dsem0)

            @pl.when(i0 >= 1)
            def _():
                decode_row(i0 - 1, 1, dsem1, osem1)

            @pl.when(i0 + 2 < tpw)
            def _():
                fetch(i0 + 2, 0, gsem0)

            fwait(1, gsem1)
            stage_and_gather(i0 + 1, 1, dsem1)
            decode_row(i0, 0, dsem0, osem0)
            return 0

        lax.fori_loop(0, tpw // 2, pair, 0)
        decode_row(tpw - 1, 1, dsem1, osem1)
        pltpu.make_async_copy(
            out_v.at[0], out_hbm.at[base], osem0).wait()
        pltpu.make_async_copy(
            out_v.at[1], out_hbm.at[base], osem1).wait()

    return fused


def _make_decode(T, C, L, NC, NS):
    NW = NC * NS
    tpw = T // NW
    assert tpw % 2 == 0 and tpw >= 4
    LANES = 16
    nch = C // LANES

    mesh = plsc.VectorSubcoreMesh(core_axis_name="c", subcore_axis_name="s")

    @functools.partial(
        pl.kernel,
        mesh=mesh,
        out_type=jax.ShapeDtypeStruct((T, C), jnp.float32),
        scratch_types=[
            pltpu.VMEM((tpw, K), jnp.int32),
            pltpu.VMEM((tpw * K,), jnp.float32),
            pltpu.VMEM((L, C), jnp.float32),
            pltpu.VMEM((2, K, C), jnp.float32),
            pltpu.VMEM((2, C), jnp.float32),
            pltpu.SemaphoreType.DMA,
            pltpu.SemaphoreType.DMA,
            pltpu.SemaphoreType.DMA,
            pltpu.SemaphoreType.DMA,
        ],
    )
    def decode(vals_hbm, idx_hbm, wdt_hbm, db_hbm, out_hbm,
               idx_v, vals_v, db_v, rows_v, out_v, gsem0, gsem1, osem0, osem1):
        wid = lax.axis_index("s") * NC + lax.axis_index("c")
        base = wid * tpw
        pltpu.sync_copy(idx_hbm.at[pl.ds(base, tpw)], idx_v)
        pltpu.sync_copy(vals_hbm.at[pl.ds(base * K, tpw * K)], vals_v)
        pltpu.sync_copy(db_hbm, db_v)

        def gather(i, buf, sem):
            pltpu.async_copy(wdt_hbm.at[idx_v.at[i]], rows_v.at[buf], sem)

        def gwait(buf, sem):
            pltpu.make_async_copy(
                wdt_hbm.at[idx_v.at[0]], rows_v.at[buf], sem).wait()

        def owait(buf, i, sem):
            pltpu.make_async_copy(
                out_v.at[buf], out_hbm.at[i], sem).wait()

        def compute(i, buf):
            lrow = lax.rem(i, L)
            dn = lax.GatherDimensionNumbers(
                offset_dims=(), collapsed_slice_dims=(0,),
                start_index_map=(0,))

            def _splat(vec, k):
                idx = jnp.full((LANES, 1), k, jnp.int32)
                return lax.gather(
                    vec, idx, dn, (1,),
                    mode=lax.GatherScatterMode.PROMISE_IN_BOUNDS)

            splats = []
            for k0 in range(0, K, LANES):
                vchunk = jnp.maximum(vals_v[pl.ds(i * K + k0, LANES)], 0.0)
                for k in range(LANES):
                    splats.append(_splat(vchunk, k))

            def col(c, _):
                sl = pl.ds(c * LANES, LANES)
                acc = db_v[lrow, sl]
                for k in range(K):
                    acc = acc + splats[k] * rows_v[buf, k, sl]
                out_v[buf, sl] = acc
                return 0

            lax.fori_loop(0, nch, col, 0)

        gather(0, 0, gsem0)

        def pair(p, _):
            i0 = 2 * p
            gather(i0 + 1, 1, gsem1)
            gwait(0, gsem0)

            @pl.when(i0 >= 2)
            def _():
                owait(0, base + i0 - 2, osem0)

            compute(i0, 0)
            pltpu.async_copy(out_v.at[0], out_hbm.at[base + i0], osem0)

            @pl.when(i0 + 2 < tpw)
            def _():
                gather(i0 + 2, 0, gsem0)

            gwait(1, gsem1)

            @pl.when(i0 >= 2)
            def _():
                owait(1, base + i0 - 1, osem1)

            compute(i0 + 1, 1)
            pltpu.async_copy(out_v.at[1], out_hbm.at[base + i0 + 1], osem1)
            return 0

        lax.fori_loop(0, tpw // 2, pair, 0)
        owait(0, base + tpw - 2, osem0)
        owait(1, base + tpw - 1, osem1)

    return decode


def kernel(x, decoder_b, encoder_w, encoder_b, decoder_w):
    B, L, C = x.shape
    H = encoder_w.shape[0]
    T = B * L

    x2 = x.reshape(T, C)
    we_t = encoder_w.T
    wd_t = decoder_w.T

    info = plsc.get_sparse_core_info()
    NHALF = 2
    Th = T // NHALF
    fused = _make_fused_sc(Th, H, C, L, info.num_cores, info.num_subcores)

    outs = []
    for h in range(NHALF):
        xh = lax.slice_in_dim(x2, h * Th, (h + 1) * Th, axis=0)
        pre = _encode(xh, decoder_b, we_t, encoder_b)
        outs.append(fused(pre, wd_t))
    out2 = jnp.concatenate(outs, axis=0)
    return out2.reshape(B, L, C) + decoder_b[None]


# speedup vs baseline: 1.0754x; 1.0489x over previous
import functools

import jax
import jax.numpy as jnp
from jax import lax
from jax.experimental import pallas as pl
from jax.experimental.pallas import tpu as pltpu
from jax.experimental.pallas import tpu_sc as plsc

K = 32


def _encode_body(x_ref, db_ref, w_ref, eb_ref, out_ref, bm_ref):
    xb = x_ref[...]
    lrep = xb.shape[0] // db_ref.shape[0]
    db = jnp.broadcast_to(db_ref[...][None], (lrep,) + db_ref.shape)
    db = db.reshape(xb.shape)
    xc = xb - db
    acc = jax.lax.dot_general(
        xc, w_ref[...], (((1,), (0,)), ((), ())),
        precision=jax.lax.Precision.DEFAULT,
        preferred_element_type=jnp.float32,
    )
    pre = acc + eb_ref[...]
    out_ref[...] = pre
    rb, hb = pre.shape
    bm_ref[...] = jnp.max(pre.reshape(rb, hb // 128, 128), axis=1)


def _encode(x2, decoder_b, we_t, encoder_b, *, rb=768, hb=2048):
    T, C = x2.shape
    H = we_t.shape[1]
    L = decoder_b.shape[0]
    eb2 = encoder_b.reshape(1, H)
    grid = (T // rb, H // hb)
    return pl.pallas_call(
        _encode_body,
        grid=grid,
        in_specs=[
            pl.BlockSpec((rb, C), lambda r, h: (r, 0)),
            pl.BlockSpec((L, C), lambda r, h: (0, 0)),
            pl.BlockSpec((C, hb), lambda r, h: (0, h)),
            pl.BlockSpec((1, hb), lambda r, h: (0, h)),
        ],
        out_specs=[
            pl.BlockSpec((rb, hb), lambda r, h: (r, h)),
            pl.BlockSpec((rb, 128), lambda r, h: (r, h)),
        ],
        out_shape=[
            jax.ShapeDtypeStruct((T, H), jnp.float32),
            jax.ShapeDtypeStruct((T, (H // hb) * 128), jnp.float32),
        ],
    )(x2, decoder_b, we_t, eb2)


def _topk_body(pre_ref, vals_ref, idx_ref, scratch):
    scratch[...] = pre_ref[...]
    vals_ref[...] = jnp.zeros_like(vals_ref)
    idx_ref[...] = jnp.zeros_like(idx_ref)
    R, H = scratch.shape

    def body(k, _):
        p = scratch[...]
        m = jnp.max(p, axis=1, keepdims=True)
        lane = jax.lax.broadcasted_iota(jnp.int32, p.shape, 1)
        j = jnp.min(jnp.where(p == m, lane, H), axis=1, keepdims=True)
        onehot = jax.lax.broadcasted_iota(jnp.int32, (R, K), 1) == k
        vals_ref[...] += jnp.where(onehot, m, 0.0)
        idx_ref[...] += jnp.where(onehot, j, 0)
        scratch[...] = jnp.where(lane == j, -jnp.inf, p)
        return 0

    lax.fori_loop(0, K, body, 0)


def _topk_tc(pre, *, rb=128):
    T, H = pre.shape
    return pl.pallas_call(
        _topk_body,
        grid=(T // rb,),
        in_specs=[pl.BlockSpec((rb, H), lambda r: (r, 0))],
        out_specs=[
            pl.BlockSpec((rb, K), lambda r: (r, 0)),
            pl.BlockSpec((rb, K), lambda r: (r, 0)),
        ],
        out_shape=[
            jax.ShapeDtypeStruct((T, K), jnp.float32),
            jax.ShapeDtypeStruct((T, K), jnp.int32),
        ],
        scratch_shapes=[pltpu.VMEM((rb, H), jnp.float32)],
    )(pre)


_NEG = -3.0e38


def _splat16(vec, lane):
    dn = lax.GatherDimensionNumbers(
        offset_dims=(), collapsed_slice_dims=(0,), start_index_map=(0,))
    idx = jnp.full((16, 1), lane, jnp.int32)
    return lax.gather(vec, idx, dn, (1,),
                      mode=lax.GatherScatterMode.PROMISE_IN_BOUNDS)


def _rev16(v):
    return lax.rev(v, dimensions=(0,))


def _merge_run(run, v):
    return jnp.sort(jnp.maximum(run, _rev16(jnp.sort(v))))


def _merge_sorted(a, b):
    return jnp.sort(jnp.maximum(a, _rev16(b)))


def _top16_static(read_fn, n):
    neg = jnp.full((16,), _NEG, jnp.float32)

    def step(s, runs):
        return tuple(
            _merge_run(runs[c], read_fn(4 * s + c)) for c in range(4))

    r0, r1, r2, r3 = lax.fori_loop(0, n // 4, step, (neg, neg, neg, neg))
    return _merge_sorted(_merge_sorted(r0, r1), _merge_sorted(r2, r3))


def _top16_dyn(read_fn, nv):
    neg = jnp.full((16,), _NEG, jnp.float32)
    return lax.fori_loop(0, nv, lambda s, r: _merge_run(r, read_fn(s)), neg)


def _merge_run_kv(rk, rv, k, v):
    sk, sv = plsc.sort_key_val(k, v)
    sk, sv = _rev16(sk), _rev16(sv)
    m = rk >= sk
    ok, ov = plsc.sort_key_val(jnp.where(m, rk, sk), jnp.where(m, rv, sv))
    return ok, ov


def _merge_sorted_kv(ak, av, bk, bv):
    bk, bv = _rev16(bk), _rev16(bv)
    m = ak >= bk
    ok, ov = plsc.sort_key_val(jnp.where(m, ak, bk), jnp.where(m, av, bv))
    return ok, ov


def _top16_kv(read_fn, n, nchain):
    neg = jnp.full((16,), _NEG, jnp.float32)
    zero = jnp.zeros((16,), jnp.int32)

    def step(s, runs):
        out = []
        for c in range(nchain):
            k, v = read_fn(nchain * s + c)
            out.append(tuple(_merge_run_kv(runs[c][0], runs[c][1], k, v)))
        return tuple(out)

    runs = lax.fori_loop(0, n // nchain, step, ((neg, zero),) * nchain)
    while len(runs) > 1:
        runs = tuple(
            tuple(_merge_sorted_kv(*runs[2 * i], *runs[2 * i + 1]))
            for i in range(len(runs) // 2))
    return runs[0]


def _make_topk_sc(T, H, NC, NS):
    NW = NC * NS
    tpw = T // NW
    assert tpw % 2 == 0 and tpw >= 4
    NGR = H // 256
    NB = H // 16

    mesh = plsc.VectorSubcoreMesh(core_axis_name="c", subcore_axis_name="s")

    @functools.partial(
        pl.kernel,
        mesh=mesh,
        out_type=[
            jax.ShapeDtypeStruct((T * K,), jnp.float32),
            jax.ShapeDtypeStruct((T * K,), jnp.int32),
        ],
        scratch_types=[
            pltpu.VMEM((2, H), jnp.float32),
            pltpu.VMEM((NB,), jnp.float32),
            pltpu.VMEM((NB,), jnp.float32),
            pltpu.VMEM((2 * K * 16,), jnp.float32),
            pltpu.VMEM((2 * K * 16,), jnp.int32),
            pltpu.VMEM((2 * K * 16,), jnp.float32),
            pltpu.VMEM((tpw * K,), jnp.float32),
            pltpu.VMEM((tpw * K,), jnp.int32),
            pltpu.SemaphoreType.DMA,
            pltpu.SemaphoreType.DMA,
        ],
        compiler_params=pltpu.CompilerParams(needs_layout_passes=False),
    )
    def topk(pre_hbm, vals_hbm, idx_hbm,
             row_v, gm_v, gm2_v, cv_v, ci_v, cv2_v,
             ov_v, oi_v, gsem0, gsem1):
        wid = lax.axis_index("s") * NC + lax.axis_index("c")
        base = wid * tpw
        iota = lax.iota(jnp.int32, 16)

        def fetch(i, buf, sem):
            pltpu.async_copy(pre_hbm.at[base + i], row_v.at[buf], sem)

        def fwait(buf, sem):
            pltpu.make_async_copy(
                pre_hbm.at[base], row_v.at[buf], sem).wait()

        def process(i, buf):
            def grp(g, _):
                m = jnp.full((16,), _NEG, jnp.float32)
                for j in range(16):
                    m = jnp.maximum(
                        m, row_v[buf, pl.ds(g * 256 + j * 16, 16)])
                gm_v[pl.ds(g * 16, 16)] = m
                return 0

            lax.fori_loop(0, NGR, grp, 0)

            def rd1(s):
                return gm_v[pl.ds(s * 16, 16)], iota + s * 16

            r1k, r1v = _top16_kv(rd1, NB // 16, 4)
            t16 = _splat16(r1k, 0)

            def rem(g, _):
                v = bm_v[buf, pl.ds(g * 16, 16)]
                gm2_v[pl.ds(g * 16, 16)] = jnp.where(v >= t16, _NEG, v)
                return 0

            lax.fori_loop(0, NB // 16, rem, 0)

            def rd2(s):
                return gm2_v[pl.ds(s * 16, 16)], iota + s * 16

            r2k, r2v = _top16_kv(rd2, NB // 16, 4)

            bufidx = jnp.full((16,), buf, jnp.int32)
            for rr in range(2 * 16):
                bidv = r1v if rr < 16 else r2v
                bid = _splat16(bidv, rr % 16)
                eidx = (jnp.right_shift(bid, 4) * 256
                        + jnp.bitwise_and(bid, 15) + iota * 16)
                candv = plsc.load_gather(row_v, [bufidx, eidx])
                cv_v[pl.ds(rr * 16, 16)] = candv
                ci_v[pl.ds(rr * 16, 16)] = eidx

            def rd3(s):
                return (cv_v[pl.ds(s * 16, 16)], ci_v[pl.ds(s * 16, 16)])

            f1k, f1v = _top16_kv(rd3, 2 * 16, 2)
            tf = _splat16(f1k, 0)

            def rem2(s, _):
                v = cv_v[pl.ds(s * 16, 16)]
                cv2_v[pl.ds(s * 16, 16)] = jnp.where(v >= tf, _NEG, v)
                return 0

            lax.fori_loop(0, 2 * 16, rem2, 0)

            def rd4(s):
                return (cv2_v[pl.ds(s * 16, 16)], ci_v[pl.ds(s * 16, 16)])

            f2k, f2v = _top16_kv(rd4, 2 * 16, 2)

            ov_v[pl.ds(i * K, 16)] = f1k
            ov_v[pl.ds(i * K + 16, 16)] = f2k
            oi_v[pl.ds(i * K, 16)] = f1v
            oi_v[pl.ds(i * K + 16, 16)] = f2v

        fetch(0, 0, gsem0)

        def pair(p, _):
            i0 = 2 * p
            fetch(i0 + 1, 1, gsem1)
            fwait(0, gsem0)
            process(i0, 0)

            @pl.when(i0 + 2 < tpw)
            def _():
                fetch(i0 + 2, 0, gsem0)

            fwait(1, gsem1)
            process(i0 + 1, 1)
            return 0

        lax.fori_loop(0, tpw // 2, pair, 0)
        pltpu.sync_copy(ov_v, vals_hbm.at[pl.ds(base * K, tpw * K)])
        pltpu.sync_copy(oi_v, idx_hbm.at[pl.ds(base * K, tpw * K)])

    return topk


def _make_fused_sc(T, H, C, L, NC, NS):
    NW = NC * NS
    tpw = T // NW
    assert tpw % 2 == 0 and tpw >= 4
    NGR = H // 256
    NB = H // 16
    NR = 2 * 16
    LANES = 16
    nch = C // LANES

    mesh = plsc.VectorSubcoreMesh(core_axis_name="c", subcore_axis_name="s")

    @functools.partial(
        pl.kernel,
        mesh=mesh,
        out_type=jax.ShapeDtypeStruct((T, C), jnp.float32),
        scratch_types=[
            pltpu.VMEM((2, H), jnp.float32),
            pltpu.VMEM((2, NB), jnp.float32),
            pltpu.VMEM((NB,), jnp.float32),
            pltpu.VMEM((NR * 16,), jnp.float32),
            pltpu.VMEM((NR * 16,), jnp.int32),
            pltpu.VMEM((NR * 16,), jnp.float32),
            pltpu.VMEM((2, 2 * K), jnp.float32),
            pltpu.VMEM((2, K), jnp.int32),
            pltpu.VMEM((2, K, C), jnp.float32),
            pltpu.VMEM((2, C), jnp.float32),
            pltpu.SemaphoreType.DMA,
            pltpu.SemaphoreType.DMA,
            pltpu.SemaphoreType.DMA,
            pltpu.SemaphoreType.DMA,
            pltpu.SemaphoreType.DMA,
            pltpu.SemaphoreType.DMA,
        ],
        compiler_params=pltpu.CompilerParams(needs_layout_passes=False),
    )
    def fused(pre_hbm, bm_hbm, wde_hbm, out_hbm,
              row_v, bm_v, gm2_v, cv_v, ci_v, cv2_v, vv_v, di_v, rows_v,
              out_v, gsem0, gsem1, dsem0, dsem1, osem0, osem1):
        wid = lax.axis_index("s") * NC + lax.axis_index("c")
        base = wid * tpw
        iota = lax.iota(jnp.int32, 16)

        def fetch(i, buf, sem):
            pltpu.async_copy(pre_hbm.at[base + i], row_v.at[buf], sem)
            pltpu.async_copy(bm_hbm.at[base + i], bm_v.at[buf], sem)

        def fwait(buf, sem):
            pltpu.make_async_copy(
                pre_hbm.at[base], row_v.at[buf], sem).wait()
            pltpu.make_async_copy(
                bm_hbm.at[base], bm_v.at[buf], sem).wait()

        def topk_row(buf):
            def rd1(s):
                return bm_v[buf, pl.ds(s * 16, 16)], iota + s * 16

            r1k, r1v = _top16_kv(rd1, NB // 16, 4)
            t16 = _splat16(r1k, 0)

            def rem(g, _):
                v = bm_v[buf, pl.ds(g * 16, 16)]
                gm2_v[pl.ds(g * 16, 16)] = jnp.where(v >= t16, _NEG, v)
                return 0

            lax.fori_loop(0, NB // 16, rem, 0)

            def rd2(s):
                return gm2_v[pl.ds(s * 16, 16)], iota + s * 16

            r2k, r2v = _top16_kv(rd2, NB // 16, 4)

            bufidx = jnp.full((16,), buf, jnp.int32)
            for rr in range(NR):
                bidv = r1v if rr < 16 else r2v
                bid = _splat16(bidv, rr % 16)
                eidx = (jnp.right_shift(bid, 7) * 2048
                        + jnp.bitwise_and(bid, 127) + iota * 128)
                cv_v[pl.ds(rr * 16, 16)] = plsc.load_gather(
                    row_v, [bufidx, eidx])
                ci_v[pl.ds(rr * 16, 16)] = eidx

            def rd3(s):
                return (cv_v[pl.ds(s * 16, 16)], ci_v[pl.ds(s * 16, 16)])

            f1k, f1v = _top16_kv(rd3, NR, 2)
            tf = _splat16(f1k, 0)

            def rem2(s, _):
                v = cv_v[pl.ds(s * 16, 16)]
                cv2_v[pl.ds(s * 16, 16)] = jnp.where(v >= tf, _NEG, v)
                return 0

            lax.fori_loop(0, NR, rem2, 0)

            def rd4(s):
                return (cv2_v[pl.ds(s * 16, 16)], ci_v[pl.ds(s * 16, 16)])

            f2k, f2v = _top16_kv(rd4, NR, 2)
            return f1k, f1v, f2k, f2v

        def stage_and_gather(i, slot, dsem):
            f1k, f1v, f2k, f2v = topk_row(slot)
            vv_v[slot, pl.ds(0, 16)] = jnp.maximum(f1k, 0.0)
            vv_v[slot, pl.ds(16, 16)] = jnp.maximum(f2k, 0.0)
            di_v[slot, pl.ds(0, 16)] = f1v
            di_v[slot, pl.ds(16, 16)] = f2v
            pltpu.async_copy(
                wde_hbm.at[di_v.at[slot]], rows_v.at[slot], dsem)

        def decode_row(i, slot, dsem, osem):
            pltpu.make_async_copy(
                wde_hbm.at[di_v.at[slot]], rows_v.at[slot], dsem).wait()

            @pl.when(i >= 2)
            def _():
                pltpu.make_async_copy(
                    out_v.at[slot], out_hbm.at[base], osem).wait()

            v0 = vv_v[slot, pl.ds(0, 16)]
            v1 = vv_v[slot, pl.ds(16, 16)]
            splats = [_splat16(v0, k) for k in range(16)]
            splats += [_splat16(v1, k) for k in range(16)]

            def col(c, _):
                sl = pl.ds(c * LANES, LANES)
                acc = splats[0] * rows_v[slot, 0, sl]
                for k in range(1, K):
                    acc = acc + splats[k] * rows_v[slot, k, sl]
                out_v[slot, sl] = acc
                return 0

            lax.fori_loop(0, nch, col, 0)
            pltpu.async_copy(out_v.at[slot], out_hbm.at[base + i], osem)

        fetch(0, 0, gsem0)

        def pair(p, _):
            i0 = 2 * p
            fetch(i0 + 1, 1, gsem1)
            fwait(0, gsem0)
            stage_and_gather(i0, 0, dsem0)

            @pl.when(i0 >= 1)
            def _():
                decode_row(i0 - 1, 1, dsem1, osem1)

            @pl.when(i0 + 2 < tpw)
            def _():
                fetch(i0 + 2, 0, gsem0)

            fwait(1, gsem1)
            stage_and_gather(i0 + 1, 1, dsem1)
            decode_row(i0, 0, dsem0, osem0)
            return 0

        lax.fori_loop(0, tpw // 2, pair, 0)
        decode_row(tpw - 1, 1, dsem1, osem1)
        pltpu.make_async_copy(
            out_v.at[0], out_hbm.at[base], osem0).wait()
        pltpu.make_async_copy(
            out_v.at[1], out_hbm.at[base], osem1).wait()

    return fused


def _make_decode(T, C, L, NC, NS):
    NW = NC * NS
    tpw = T // NW
    assert tpw % 2 == 0 and tpw >= 4
    LANES = 16
    nch = C // LANES

    mesh = plsc.VectorSubcoreMesh(core_axis_name="c", subcore_axis_name="s")

    @functools.partial(
        pl.kernel,
        mesh=mesh,
        out_type=jax.ShapeDtypeStruct((T, C), jnp.float32),
        scratch_types=[
            pltpu.VMEM((tpw, K), jnp.int32),
            pltpu.VMEM((tpw * K,), jnp.float32),
            pltpu.VMEM((L, C), jnp.float32),
            pltpu.VMEM((2, K, C), jnp.float32),
            pltpu.VMEM((2, C), jnp.float32),
            pltpu.SemaphoreType.DMA,
            pltpu.SemaphoreType.DMA,
            pltpu.SemaphoreType.DMA,
            pltpu.SemaphoreType.DMA,
        ],
    )
    def decode(vals_hbm, idx_hbm, wdt_hbm, db_hbm, out_hbm,
               idx_v, vals_v, db_v, rows_v, out_v, gsem0, gsem1, osem0, osem1):
        wid = lax.axis_index("s") * NC + lax.axis_index("c")
        base = wid * tpw
        pltpu.sync_copy(idx_hbm.at[pl.ds(base, tpw)], idx_v)
        pltpu.sync_copy(vals_hbm.at[pl.ds(base * K, tpw * K)], vals_v)
        pltpu.sync_copy(db_hbm, db_v)

        def gather(i, buf, sem):
            pltpu.async_copy(wdt_hbm.at[idx_v.at[i]], rows_v.at[buf], sem)

        def gwait(buf, sem):
            pltpu.make_async_copy(
                wdt_hbm.at[idx_v.at[0]], rows_v.at[buf], sem).wait()

        def owait(buf, i, sem):
            pltpu.make_async_copy(
                out_v.at[buf], out_hbm.at[i], sem).wait()

        def compute(i, buf):
            lrow = lax.rem(i, L)
            dn = lax.GatherDimensionNumbers(
                offset_dims=(), collapsed_slice_dims=(0,),
                start_index_map=(0,))

            def _splat(vec, k):
                idx = jnp.full((LANES, 1), k, jnp.int32)
                return lax.gather(
                    vec, idx, dn, (1,),
                    mode=lax.GatherScatterMode.PROMISE_IN_BOUNDS)

            splats = []
            for k0 in range(0, K, LANES):
                vchunk = jnp.maximum(vals_v[pl.ds(i * K + k0, LANES)], 0.0)
                for k in range(LANES):
                    splats.append(_splat(vchunk, k))

            def col(c, _):
                sl = pl.ds(c * LANES, LANES)
                acc = db_v[lrow, sl]
                for k in range(K):
                    acc = acc + splats[k] * rows_v[buf, k, sl]
                out_v[buf, sl] = acc
                return 0

            lax.fori_loop(0, nch, col, 0)

        gather(0, 0, gsem0)

        def pair(p, _):
            i0 = 2 * p
            gather(i0 + 1, 1, gsem1)
            gwait(0, gsem0)

            @pl.when(i0 >= 2)
            def _():
                owait(0, base + i0 - 2, osem0)

            compute(i0, 0)
            pltpu.async_copy(out_v.at[0], out_hbm.at[base + i0], osem0)

            @pl.when(i0 + 2 < tpw)
            def _():
                gather(i0 + 2, 0, gsem0)

            gwait(1, gsem1)

            @pl.when(i0 >= 2)
            def _():
                owait(1, base + i0 - 1, osem1)

            compute(i0 + 1, 1)
            pltpu.async_copy(out_v.at[1], out_hbm.at[base + i0 + 1], osem1)
            return 0

        lax.fori_loop(0, tpw // 2, pair, 0)
        owait(0, base + tpw - 2, osem0)
        owait(1, base + tpw - 1, osem1)

    return decode


def kernel(x, decoder_b, encoder_w, encoder_b, decoder_w):
    B, L, C = x.shape
    H = encoder_w.shape[0]
    T = B * L

    x2 = x.reshape(T, C)
    we_t = encoder_w.T
    wd_t = decoder_w.T

    info = plsc.get_sparse_core_info()
    NHALF = 2
    Th = T // NHALF
    fused = _make_fused_sc(Th, H, C, L, info.num_cores, info.num_subcores)

    outs = []
    for h in range(NHALF):
        xh = lax.slice_in_dim(x2, h * Th, (h + 1) * Th, axis=0)
        pre, bm = _encode(xh, decoder_b, we_t, encoder_b)
        outs.append(fused(pre, bm, wd_t))
    out2 = jnp.concatenate(outs, axis=0)
    return out2.reshape(B, L, C) + decoder_b[None]


# speedup vs baseline: 1.0757x; 1.0002x over previous
import functools

import jax
import jax.numpy as jnp
from jax import lax
from jax.experimental import pallas as pl
from jax.experimental.pallas import tpu as pltpu
from jax.experimental.pallas import tpu_sc as plsc

K = 32


def _encode_body(x_ref, db_ref, w_ref, eb_ref, out_ref, bm_ref):
    xb = x_ref[...]
    lrep = xb.shape[0] // db_ref.shape[0]
    db = jnp.broadcast_to(db_ref[...][None], (lrep,) + db_ref.shape)
    db = db.reshape(xb.shape)
    xc = xb - db
    acc = jax.lax.dot_general(
        xc, w_ref[...], (((1,), (0,)), ((), ())),
        precision=jax.lax.Precision.DEFAULT,
        preferred_element_type=jnp.float32,
    )
    pre = acc + eb_ref[...]
    out_ref[...] = pre
    rb, hb = pre.shape
    bm_ref[...] = jnp.max(pre.reshape(rb, hb // 128, 128), axis=1)


def _encode(x2, decoder_b, we_t, encoder_b, *, rb=768, hb=2048):
    T, C = x2.shape
    H = we_t.shape[1]
    L = decoder_b.shape[0]
    eb2 = encoder_b.reshape(1, H)
    grid = (T // rb, H // hb)
    return pl.pallas_call(
        _encode_body,
        grid=grid,
        in_specs=[
            pl.BlockSpec((rb, C), lambda r, h: (r, 0)),
            pl.BlockSpec((L, C), lambda r, h: (0, 0)),
            pl.BlockSpec((C, hb), lambda r, h: (0, h)),
            pl.BlockSpec((1, hb), lambda r, h: (0, h)),
        ],
        out_specs=[
            pl.BlockSpec((rb, hb), lambda r, h: (r, h)),
            pl.BlockSpec((rb, 128), lambda r, h: (r, h)),
        ],
        out_shape=[
            jax.ShapeDtypeStruct((T, H), jnp.float32),
            jax.ShapeDtypeStruct((T, (H // hb) * 128), jnp.float32),
        ],
    )(x2, decoder_b, we_t, eb2)


_NEG = -3.0e38


def _splat16(vec, lane):
    dn = lax.GatherDimensionNumbers(
        offset_dims=(), collapsed_slice_dims=(0,), start_index_map=(0,))
    idx = jnp.full((16, 1), lane, jnp.int32)
    return lax.gather(vec, idx, dn, (1,),
                      mode=lax.GatherScatterMode.PROMISE_IN_BOUNDS)


def _rev16(v):
    return lax.rev(v, dimensions=(0,))


def _merge_run_kv(rk, rv, k, v):
    sk, sv = plsc.sort_key_val(k, v)
    sk, sv = _rev16(sk), _rev16(sv)
    m = rk >= sk
    ok, ov = plsc.sort_key_val(jnp.where(m, rk, sk), jnp.where(m, rv, sv))
    return ok, ov


def _merge_sorted_kv(ak, av, bk, bv):
    bk, bv = _rev16(bk), _rev16(bv)
    m = ak >= bk
    ok, ov = plsc.sort_key_val(jnp.where(m, ak, bk), jnp.where(m, av, bv))
    return ok, ov


def _top16_kv(read_fn, n, nchain):
    neg = jnp.full((16,), _NEG, jnp.float32)
    zero = jnp.zeros((16,), jnp.int32)

    def step(s, runs):
        out = []
        for c in range(nchain):
            k, v = read_fn(nchain * s + c)
            out.append(tuple(_merge_run_kv(runs[c][0], runs[c][1], k, v)))
        return tuple(out)

    runs = lax.fori_loop(0, n // nchain, step, ((neg, zero),) * nchain)
    while len(runs) > 1:
        runs = tuple(
            tuple(_merge_sorted_kv(*runs[2 * i], *runs[2 * i + 1]))
            for i in range(len(runs) // 2))
    return runs[0]


def _make_fused_sc(T, H, C, L, NC, NS):
    NW = NC * NS
    tpw = T // NW
    assert tpw % 2 == 0 and tpw >= 4
    NGR = H // 256
    NB = H // 16
    NR = 2 * 16
    LANES = 16
    nch = C // LANES

    mesh = plsc.VectorSubcoreMesh(core_axis_name="c", subcore_axis_name="s")

    @functools.partial(
        pl.kernel,
        mesh=mesh,
        out_type=jax.ShapeDtypeStruct((T, C), jnp.float32),
        scratch_types=[
            pltpu.VMEM((2, H), jnp.float32),
            pltpu.VMEM((2, NB), jnp.float32),
            pltpu.VMEM((NB,), jnp.float32),
            pltpu.VMEM((NR * 16,), jnp.float32),
            pltpu.VMEM((NR * 16,), jnp.int32),
            pltpu.VMEM((NR * 16,), jnp.float32),
            pltpu.VMEM((2, 2 * K), jnp.float32),
            pltpu.VMEM((2, K), jnp.int32),
            pltpu.VMEM((2, K, C), jnp.float32),
            pltpu.VMEM((2, C), jnp.float32),
            pltpu.SemaphoreType.DMA,
            pltpu.SemaphoreType.DMA,
            pltpu.SemaphoreType.DMA,
            pltpu.SemaphoreType.DMA,
            pltpu.SemaphoreType.DMA,
            pltpu.SemaphoreType.DMA,
        ],
        compiler_params=pltpu.CompilerParams(needs_layout_passes=False),
    )
    def fused(pre_hbm, bm_hbm, wde_hbm, out_hbm,
              row_v, bm_v, gm2_v, cv_v, ci_v, cv2_v, vv_v, di_v, rows_v,
              out_v, gsem0, gsem1, dsem0, dsem1, osem0, osem1):
        wid = lax.axis_index("s") * NC + lax.axis_index("c")
        base = wid * tpw
        iota = lax.iota(jnp.int32, 16)

        def fetch(i, buf, sem):
            pltpu.async_copy(pre_hbm.at[base + i], row_v.at[buf], sem)
            pltpu.async_copy(bm_hbm.at[base + i], bm_v.at[buf], sem)

        def fwait(buf, sem):
            pltpu.make_async_copy(
                pre_hbm.at[base], row_v.at[buf], sem).wait()
            pltpu.make_async_copy(
                bm_hbm.at[base], bm_v.at[buf], sem).wait()

        def topk_row(buf):
            def rd1(s):
                return bm_v[buf, pl.ds(s * 16, 16)], iota + s * 16

            r1k, r1v = _top16_kv(rd1, NB // 16, 4)
            t16 = _splat16(r1k, 0)

            def rem(g, _):
                v = bm_v[buf, pl.ds(g * 16, 16)]
                gm2_v[pl.ds(g * 16, 16)] = jnp.where(v >= t16, _NEG, v)
                return 0

            lax.fori_loop(0, NB // 16, rem, 0)

            def rd2(s):
                return gm2_v[pl.ds(s * 16, 16)], iota + s * 16

            r2k, r2v = _top16_kv(rd2, NB // 16, 4)

            bufidx = jnp.full((16,), buf, jnp.int32)
            for rr in range(NR):
                bidv = r1v if rr < 16 else r2v
                bid = _splat16(bidv, rr % 16)
                eidx = (jnp.right_shift(bid, 7) * 2048
                        + jnp.bitwise_and(bid, 127) + iota * 128)
                cv_v[pl.ds(rr * 16, 16)] = plsc.load_gather(
                    row_v, [bufidx, eidx])
                ci_v[pl.ds(rr * 16, 16)] = eidx

            def rd3(s):
                return (cv_v[pl.ds(s * 16, 16)], ci_v[pl.ds(s * 16, 16)])

            f1k, f1v = _top16_kv(rd3, NR, 2)
            tf = _splat16(f1k, 0)

            def rem2(s, _):
                v = cv_v[pl.ds(s * 16, 16)]
                cv2_v[pl.ds(s * 16, 16)] = jnp.where(v >= tf, _NEG, v)
                return 0

            lax.fori_loop(0, NR, rem2, 0)

            def rd4(s):
                return (cv2_v[pl.ds(s * 16, 16)], ci_v[pl.ds(s * 16, 16)])

            f2k, f2v = _top16_kv(rd4, NR, 2)
            return f1k, f1v, f2k, f2v

        def stage_and_gather(i, slot, dsem):
            f1k, f1v, f2k, f2v = topk_row(slot)
            vv_v[slot, pl.ds(0, 16)] = jnp.maximum(f1k, 0.0)
            vv_v[slot, pl.ds(16, 16)] = jnp.maximum(f2k, 0.0)
            di_v[slot, pl.ds(0, 16)] = f1v
            di_v[slot, pl.ds(16, 16)] = f2v
            pltpu.async_copy(
                wde_hbm.at[di_v.at[slot]], rows_v.at[slot], dsem)

        def decode_row(i, slot, dsem, osem):
            pltpu.make_async_copy(
                wde_hbm.at[di_v.at[slot]], rows_v.at[slot], dsem).wait()

            @pl.when(i >= 2)
            def _():
                pltpu.make_async_copy(
                    out_v.at[slot], out_hbm.at[base], osem).wait()

            v0 = vv_v[slot, pl.ds(0, 16)]
            v1 = vv_v[slot, pl.ds(16, 16)]
            splats = [_splat16(v0, k) for k in range(16)]
            splats += [_splat16(v1, k) for k in range(16)]

            def col(c, _):
                sl = pl.ds(c * LANES, LANES)
                acc = splats[0] * rows_v[slot, 0, sl]
                for k in range(1, K):
                    acc = acc + splats[k] * rows_v[slot, k, sl]
                out_v[slot, sl] = acc
                return 0

            lax.fori_loop(0, nch, col, 0)
            pltpu.async_copy(out_v.at[slot], out_hbm.at[base + i], osem)

        fetch(0, 0, gsem0)

        def pair(p, _):
            i0 = 2 * p
            fetch(i0 + 1, 1, gsem1)
            fwait(0, gsem0)
            stage_and_gather(i0, 0, dsem0)

            @pl.when(i0 >= 1)
            def _():
                decode_row(i0 - 1, 1, dsem1, osem1)

            @pl.when(i0 + 2 < tpw)
            def _():
                fetch(i0 + 2, 0, gsem0)

            fwait(1, gsem1)
            stage_and_gather(i0 + 1, 1, dsem1)
            decode_row(i0, 0, dsem0, osem0)
            return 0

        lax.fori_loop(0, tpw // 2, pair, 0)
        decode_row(tpw - 1, 1, dsem1, osem1)
        pltpu.make_async_copy(
            out_v.at[0], out_hbm.at[base], osem0).wait()
        pltpu.make_async_copy(
            out_v.at[1], out_hbm.at[base], osem1).wait()

    return fused


def kernel(x, decoder_b, encoder_w, encoder_b, decoder_w):
    B, L, C = x.shape
    H = encoder_w.shape[0]
    T = B * L

    x2 = x.reshape(T, C)
    we_t = encoder_w.T
    wd_t = decoder_w.T

    info = plsc.get_sparse_core_info()
    NHALF = 2
    Th = T // NHALF
    fused = _make_fused_sc(Th, H, C, L, info.num_cores, info.num_subcores)

    outs = []
    for h in range(NHALF):
        xh = lax.slice_in_dim(x2, h * Th, (h + 1) * Th, axis=0)
        pre, bm = _encode(xh, decoder_b, we_t, encoder_b)
        outs.append(fused(pre, bm, wd_t))
    out2 = jnp.concatenate(outs, axis=0)
    return out2.reshape(B, L, C) + decoder_b[None]
